# Initial kernel scaffold; baseline (speedup 1.0000x reference)
import jax
import jax.numpy as jnp
from jax.experimental import pallas as pl


def kernel(feat, delay, bit_position, pi_w1, pi_b1, pi_w2, pi_b2, gate_w1, gate_b1, gate_w2, gate_b2, mod_w1, mod_b1, mod_w2, mod_b2, glob_w1, glob_b1, glob_w2, glob_b2, out_w1, out_b1, out_w2, out_b2, edge_index, is_po, is_module):
    raise NotImplementedError("write your pallas kernel here")


# speedup vs baseline: 10.3805x; 10.3805x over previous
import functools

import jax
import jax.numpy as jnp
from jax import lax
from jax.experimental import pallas as pl
from jax.experimental.pallas import tpu as pltpu
from jax.experimental.pallas import tpu_sc as plsc

N = 50000
E = 800000
H = 64
F = 128

NB = 25
BN = N // NB
NSC = 2
NT = 16
CH = 2000
EPT = E // NT
NCHUNK = EPT // CH
RPT = 3128
NP = RPT * NT
GW = 16


def _leaky(x):
    return jnp.where(x >= 0, x, 0.1 * x)


def _node_embed_body(d_ref, pw1, pb1, pw2, gw1, gb1, gw2, pb2, gb2,
                     h0_ref, hg_ref, gmx_ref):
    i = pl.program_id(0)
    d = d_ref[...]
    h0 = jnp.dot(_leaky(d * pw1[...] + pb1[...]), pw2[...],
                 preferred_element_type=jnp.float32) + pb2[...]
    hg = jnp.dot(_leaky(d * gw1[...] + gb1[...]), gw2[...],
                 preferred_element_type=jnp.float32) + gb2[...]
    h0_ref[...] = h0
    hg_ref[...] = hg
    bmx = jnp.max(h0, axis=0, keepdims=True)

    @pl.when(i == 0)
    def _():
        gmx_ref[...] = bmx

    @pl.when(i > 0)
    def _():
        gmx_ref[...] = jnp.maximum(gmx_ref[...], bmx)


def _node_embed(delay, pi_w1, pi_b1, pi_w2, pi_b2,
                glob_w1, glob_b1, glob_w2, glob_b2):
    full2 = lambda s: pl.BlockSpec(s, lambda i: (0, 0))
    return pl.pallas_call(
        _node_embed_body,
        grid=(NB,),
        in_specs=[
            pl.BlockSpec((BN, 1), lambda i: (i, 0)),
            full2((1, 32)), full2((1, 32)), full2((32, H)),
            full2((1, 32)), full2((1, 32)), full2((32, H)),
            full2((1, H)), full2((1, H)),
        ],
        out_specs=[
            pl.BlockSpec((BN, H), lambda i: (i, 0)),
            pl.BlockSpec((BN, H), lambda i: (i, 0)),
            pl.BlockSpec((1, H), lambda i: (0, 0)),
        ],
        out_shape=[
            jax.ShapeDtypeStruct((N, H), jnp.float32),
            jax.ShapeDtypeStruct((N, H), jnp.float32),
            jax.ShapeDtypeStruct((1, H), jnp.float32),
        ],
    )(delay, pi_w1, pi_b1.reshape(1, 32), pi_w2,
      glob_w1, glob_b1.reshape(1, 32), glob_w2,
      pi_b2.reshape(1, H), glob_b2.reshape(1, H))


def _tables_body(h0_ref, gmx_ref, *gs):
    h0 = h0_ref[...]
    e0 = jnp.exp(h0 - gmx_ref[...])
    e1 = e0 * h0
    for j in range(4):
        gs[j][...] = e0[:, 16 * j:16 * j + 16]
        gs[4 + j][...] = e1[:, 16 * j:16 * j + 16]


def _tables(h0, gmx):
    blk = pl.BlockSpec((BN, GW), lambda i: (i, 0))
    return pl.pallas_call(
        _tables_body,
        grid=(NB,),
        in_specs=[pl.BlockSpec((BN, H), lambda i: (i, 0)),
                  pl.BlockSpec((1, H), lambda i: (0, 0))],
        out_specs=[blk] * 8,
        out_shape=[jax.ShapeDtypeStruct((N, GW), jnp.float32)] * 8,
    )(h0, gmx)


def _bit_body(b_ref, eb_ref, ebb_ref):
    b = b_ref[...]
    eb = jnp.exp(b)
    eb_ref[...] = eb
    ebb_ref[...] = eb * b


def _bit_tables(bit2d):
    r, c = bit2d.shape
    blk = pl.BlockSpec((r, c), lambda: (0, 0))
    return pl.pallas_call(
        _bit_body,
        in_specs=[blk],
        out_specs=[blk, blk],
        out_shape=[jax.ShapeDtypeStruct((r, c), jnp.float32)] * 2,
    )(bit2d)


def _edge_body(src_hbm, dst_hbm, g0, g1, g2, g3, g4, g5, g6, g7,
               b16_hbm, zg,
               o0, o1, o2, o3, o4, o5, o6, o7, q_out,
               idx_s, idx_d, rows, acc, sem):
    c = lax.axis_index("c")
    s = lax.axis_index("s")
    row0 = s * RPT

    def do_round(tbl, out_ref, linear):
        pltpu.sync_copy(zg, acc.at[pl.ds(row0, RPT)])
        plsc.subcore_barrier()

        base0 = s * EPT

        def chunk(k, carry):
            b = base0 + k * CH
            pltpu.sync_copy(dst_hbm.at[pl.ds(b, CH)], idx_d)
            if linear:
                pltpu.sync_copy(tbl.at[pl.ds(b, CH)], rows)
            else:
                pltpu.sync_copy(src_hbm.at[pl.ds(b, CH)], idx_s)
                pltpu.async_copy(tbl.at[idx_s], rows, sem).wait()
            pltpu.sync_copy(rows, acc.at[idx_d], add=True)
            return carry

        lax.fori_loop(0, NCHUNK, chunk, 0)
        plsc.subcore_barrier()
        pltpu.sync_copy(acc.at[pl.ds(row0, RPT)], out_ref.at[pl.ds(row0, RPT)])
        plsc.subcore_barrier()

    @pl.when(c == 0)
    def _():
        do_round(g0, o0, False)
        do_round(g1, o1, False)
        do_round(g2, o2, False)
        do_round(g3, o3, False)

    @pl.when(c == 1)
    def _():
        do_round(g4, o4, False)
        do_round(g5, o5, False)
        do_round(g6, o6, False)
        do_round(g7, o7, False)
        do_round(b16_hbm, q_out, True)


def _edge_pass(src, dst, gs, b16):
    zg = jnp.zeros((RPT, GW), jnp.float32)
    mesh = plsc.VectorSubcoreMesh(core_axis_name="c", subcore_axis_name="s",
                                  num_cores=NSC, num_subcores=NT)
    fn = pl.kernel(
        _edge_body,
        out_type=[jax.ShapeDtypeStruct((NP, GW), jnp.float32)] * 9,
        mesh=mesh,
        scratch_types=[
            pltpu.VMEM((CH,), jnp.int32),
            pltpu.VMEM((CH,), jnp.int32),
            pltpu.VMEM((CH, GW), jnp.float32),
            pltpu.VMEM_SHARED((NP, GW), jnp.float32),
            pltpu.SemaphoreType.DMA,
        ],
        compiler_params=pltpu.CompilerParams(use_tc_tiling_on_sc=False),
    )
    return fn(src, dst, *gs, b16, zg)


def _epilogue_body(a0, a1, a2, a3, a4, a5, a6, a7, q,
                   feat_ref, h0_ref, hg_ref, ipo, imod,
                   gw1, gb1, gw2, gb2, mw1, mb1, mw2, mb2,
                   ow1, ob1, ow2, ob2, out_ref):
    s64 = jnp.concatenate([a0[...], a1[...], a2[...], a3[...]], axis=1)
    t64 = jnp.concatenate([a4[...], a5[...], a6[...], a7[...]], axis=1)
    ng = t64 / (s64 + 1e-9)
    qq = q[...]
    sb = qq[:, 0:1]
    nb = qq[:, 1:2] / (sb + 1e-9)
    feat = feat_ref[...]

    xg = jnp.concatenate([ng, feat], axis=1)
    hgate = jnp.dot(_leaky(jnp.dot(xg, gw1[...],
                                   preferred_element_type=jnp.float32)
                           + gb1[...]), gw2[...],
                    preferred_element_type=jnp.float32) + gb2[...]
    xm = jnp.concatenate([ng, nb, feat], axis=1)
    hmod = jnp.dot(_leaky(jnp.dot(xm, mw1[...],
                                  preferred_element_type=jnp.float32)
                          + mb1[...]), mw2[...],
                   preferred_element_type=jnp.float32) + mb2[...]
    not_po = ipo[...] != 1
    hgate = jnp.where(not_po, jnp.maximum(hgate, 0.0), hgate)
    hmod = jnp.where(not_po, jnp.maximum(hmod, 0.0), hmod)
    h = jnp.where(imod[...] == 1, hmod, hgate)
    h = jnp.where(sb == 0, h0_ref[...], h)
    xo = jnp.concatenate([h, hg_ref[...]], axis=1)
    out_ref[...] = jnp.dot(_leaky(jnp.dot(xo, ow1[...],
                                          preferred_element_type=jnp.float32)
                                  + ob1[...]), ow2[...],
                           preferred_element_type=jnp.float32) + ob2[...]


def _epilogue(os_, q, feat, h0, hg, is_po, is_module,
              gate_w1, gate_b1, gate_w2, gate_b2,
              mod_w1, mod_b1, mod_w2, mod_b2,
              out_w1, out_b1, out_w2, out_b2):
    bg = pl.BlockSpec((BN, GW), lambda i: (i, 0))
    full2 = lambda a: pl.BlockSpec(a.shape, lambda i: (0, 0))
    ws = [gate_w1, gate_b1.reshape(1, -1), gate_w2, gate_b2.reshape(1, -1),
          mod_w1, mod_b1.reshape(1, -1), mod_w2, mod_b2.reshape(1, -1),
          out_w1, out_b1.reshape(1, -1), out_w2, out_b2.reshape(1, -1)]
    return pl.pallas_call(
        _epilogue_body,
        grid=(NB,),
        in_specs=[bg] * 8
        + [pl.BlockSpec((BN, GW), lambda i: (i, 0)),
                  pl.BlockSpec((BN, F), lambda i: (i, 0)),
                  pl.BlockSpec((BN, H), lambda i: (i, 0)),
                  pl.BlockSpec((BN, H), lambda i: (i, 0)),
                  pl.BlockSpec((BN, 1), lambda i: (i, 0)),
                  pl.BlockSpec((BN, 1), lambda i: (i, 0))]
        + [full2(a) for a in ws],
        out_specs=pl.BlockSpec((BN, 1), lambda i: (i, 0)),
        out_shape=jax.ShapeDtypeStruct((N, 1), jnp.float32),
    )(*os_, q, feat, h0, hg,
      is_po.reshape(N, 1), is_module.reshape(N, 1), *ws)


def kernel(feat, delay, bit_position, pi_w1, pi_b1, pi_w2, pi_b2,
           gate_w1, gate_b1, gate_w2, gate_b2,
           mod_w1, mod_b1, mod_w2, mod_b2,
           glob_w1, glob_b1, glob_w2, glob_b2,
           out_w1, out_b1, out_w2, out_b2,
           edge_index, is_po, is_module):
    src = edge_index[0]
    dst = edge_index[1]

    h0, hg, gmx = _node_embed(delay, pi_w1, pi_b1, pi_w2, pi_b2,
                              glob_w1, glob_b1, glob_w2, glob_b2)
    gs = _tables(h0, gmx)
    eb, ebb = _bit_tables(bit_position.reshape(E // 128, 128))
    b16 = jnp.concatenate(
        [eb.reshape(E, 1), ebb.reshape(E, 1),
         jnp.zeros((E, GW - 2), jnp.float32)], axis=1)

    *os_, q = _edge_pass(src, dst, gs, b16)

    return _epilogue(os_, q, feat, h0, hg, is_po, is_module,
                     gate_w1, gate_b1, gate_w2, gate_b2,
                     mod_w1, mod_b1, mod_w2, mod_b2,
                     out_w1, out_b1, out_w2, out_b2)


# speedup vs baseline: 16.4064x; 1.5805x over previous
import functools

import jax
import jax.numpy as jnp
from jax import lax
from jax.experimental import pallas as pl
from jax.experimental.pallas import tpu as pltpu
from jax.experimental.pallas import tpu_sc as plsc

N = 50000
E = 800000
H = 64
F = 128

NB = 25
BN = N // NB
NSC = 2
NT = 16
CH = 1000
EPT = E // NT
NCHUNK = EPT // CH
RPT = 3128
NP = RPT * NT
GW = 16
NBUF = 3


def _leaky(x):
    return jnp.where(x >= 0, x, 0.1 * x)


def _node_embed_body(d_ref, pw1, pb1, pw2, gw1, gb1, gw2, pb2, gb2,
                     h0_ref, hg_ref, gmx_ref):
    i = pl.program_id(0)
    d = d_ref[...]
    h0 = jnp.dot(_leaky(d * pw1[...] + pb1[...]), pw2[...],
                 preferred_element_type=jnp.float32) + pb2[...]
    hg = jnp.dot(_leaky(d * gw1[...] + gb1[...]), gw2[...],
                 preferred_element_type=jnp.float32) + gb2[...]
    h0_ref[...] = h0
    hg_ref[...] = hg
    bmx = jnp.max(h0, axis=0, keepdims=True)

    @pl.when(i == 0)
    def _():
        gmx_ref[...] = bmx

    @pl.when(i > 0)
    def _():
        gmx_ref[...] = jnp.maximum(gmx_ref[...], bmx)


def _node_embed(delay, pi_w1, pi_b1, pi_w2, pi_b2,
                glob_w1, glob_b1, glob_w2, glob_b2):
    full2 = lambda s: pl.BlockSpec(s, lambda i: (0, 0))
    return pl.pallas_call(
        _node_embed_body,
        grid=(NB,),
        in_specs=[
            pl.BlockSpec((BN, 1), lambda i: (i, 0)),
            full2((1, 32)), full2((1, 32)), full2((32, H)),
            full2((1, 32)), full2((1, 32)), full2((32, H)),
            full2((1, H)), full2((1, H)),
        ],
        out_specs=[
            pl.BlockSpec((BN, H), lambda i: (i, 0)),
            pl.BlockSpec((BN, H), lambda i: (i, 0)),
            pl.BlockSpec((1, H), lambda i: (0, 0)),
        ],
        out_shape=[
            jax.ShapeDtypeStruct((N, H), jnp.float32),
            jax.ShapeDtypeStruct((N, H), jnp.float32),
            jax.ShapeDtypeStruct((1, H), jnp.float32),
        ],
    )(delay, pi_w1, pi_b1.reshape(1, 32), pi_w2,
      glob_w1, glob_b1.reshape(1, 32), glob_w2,
      pi_b2.reshape(1, H), glob_b2.reshape(1, H))


def _tables_body(h0_ref, gmx_ref, *gs):
    h0 = h0_ref[...]
    e0 = jnp.exp(h0 - gmx_ref[...])
    e1 = e0 * h0
    for j in range(4):
        gs[j][...] = e0[:, 16 * j:16 * j + 16]
        gs[4 + j][...] = e1[:, 16 * j:16 * j + 16]


def _tables(h0, gmx):
    blk = pl.BlockSpec((BN, GW), lambda i: (i, 0))
    return pl.pallas_call(
        _tables_body,
        grid=(NB,),
        in_specs=[pl.BlockSpec((BN, H), lambda i: (i, 0)),
                  pl.BlockSpec((1, H), lambda i: (0, 0))],
        out_specs=[blk] * 8,
        out_shape=[jax.ShapeDtypeStruct((N, GW), jnp.float32)] * 8,
    )(h0, gmx)


def _edge_body(src_hbm, dst_hbm, g0, g1, g2, g3, g4, g5, g6, g7,
               bit_hbm, zg,
               o0, o1, o2, o3, o4, o5, o6, o7, q_out,
               idx_s, idx_d, rows, bitv, acc, semi, semg, sems):
    c = lax.axis_index("c")
    s = lax.axis_index("s")
    row0 = s * RPT
    base0 = s * EPT

    def do_round(tbl, out_ref, linear):
        pltpu.sync_copy(zg, acc.at[pl.ds(row0, RPT)])
        if linear:
            def zrow(i, carry):
                for b in range(NBUF):
                    rows[b][i, :] = jnp.zeros((GW,), jnp.float32)
                return carry
            lax.fori_loop(0, CH, zrow, 0)
        plsc.subcore_barrier()

        def loads(k, b):
            off = base0 + k * CH
            if not linear:
                pltpu.async_copy(src_hbm.at[pl.ds(off, CH)], idx_s[b], semi[b])
            else:
                pltpu.async_copy(bit_hbm.at[pl.ds(off, CH)], bitv[b], semi[b])
            pltpu.async_copy(dst_hbm.at[pl.ds(off, CH)], idx_d[b], semi[b])

        def fill(k, b):
            pltpu.make_async_copy(dst_hbm.at[pl.ds(0, CH)], idx_d[b],
                                  semi[b]).wait()
            if not linear:
                pltpu.make_async_copy(src_hbm.at[pl.ds(0, CH)], idx_s[b],
                                      semi[b]).wait()
                pltpu.async_copy(tbl.at[idx_s[b]], rows[b], semg[b])
            else:
                pltpu.make_async_copy(bit_hbm.at[pl.ds(0, CH)], bitv[b],
                                      semi[b]).wait()

                def bexp(i, carry):
                    v = bitv[b][pl.ds(i * 16, 16)]
                    ev = jnp.exp(v)
                    lanes = lax.iota(jnp.int32, 16) + i * 16
                    z16 = jnp.zeros((16,), jnp.int32)
                    plsc.store_scatter(rows[b], [lanes, z16], ev)
                    plsc.store_scatter(rows[b], [lanes, z16 + 1], ev * v)
                    return carry

                lax.fori_loop(0, CH // 16, bexp, 0)

        def wait_rows(b):
            if not linear:
                pltpu.make_async_copy(tbl.at[idx_s[b]], rows[b],
                                      semg[b]).wait()

        def scatter(b):
            pltpu.async_copy(rows[b], acc.at[idx_d[b]], sems[b], add=True)

        def wait_scatter(b):
            pltpu.make_async_copy(rows[b], acc.at[idx_d[b]], sems[b]).wait()

        for b in range(NBUF):
            loads(b, b)
        for b in range(NBUF):
            fill(b, b)

        def step(i, carry):
            k0 = i * NBUF
            for b in range(NBUF):
                k = k0 + b

                @pl.when(k < NCHUNK)
                def _():
                    wait_rows(b)
                    scatter(b)

                @pl.when(k < NCHUNK - NBUF)
                def _():
                    wait_scatter(b)
                    loads(k + NBUF, b)
                    fill(k + NBUF, b)
            return carry

        lax.fori_loop(0, (NCHUNK + NBUF - 1) // NBUF, step, 0)
        for b in range(NBUF):
            wait_scatter(b)
        plsc.subcore_barrier()
        pltpu.sync_copy(acc.at[pl.ds(row0, RPT)], out_ref.at[pl.ds(row0, RPT)])
        plsc.subcore_barrier()

    @pl.when(c == 0)
    def _():
        do_round(g0, o0, False)
        do_round(g1, o1, False)
        do_round(g2, o2, False)
        do_round(g3, o3, False)
        do_round(g4, o4, False)

    @pl.when(c == 1)
    def _():
        do_round(g5, o5, False)
        do_round(g6, o6, False)
        do_round(g7, o7, False)
        do_round(bit_hbm, q_out, True)


def _edge_pass(src, dst, gs, bits):
    zg = jnp.zeros((RPT, GW), jnp.float32)
    mesh = plsc.VectorSubcoreMesh(core_axis_name="c", subcore_axis_name="s",
                                  num_cores=NSC, num_subcores=NT)
    fn = pl.kernel(
        _edge_body,
        out_type=[jax.ShapeDtypeStruct((NP, GW), jnp.float32)] * 9,
        mesh=mesh,
        scratch_types=[
            [pltpu.VMEM((CH,), jnp.int32) for _ in range(NBUF)],
            [pltpu.VMEM((CH,), jnp.int32) for _ in range(NBUF)],
            [pltpu.VMEM((CH, GW), jnp.float32) for _ in range(NBUF)],
            [pltpu.VMEM((CH,), jnp.float32) for _ in range(NBUF)],
            pltpu.VMEM_SHARED((NP, GW), jnp.float32),
            [pltpu.SemaphoreType.DMA for _ in range(NBUF)],
            [pltpu.SemaphoreType.DMA for _ in range(NBUF)],
            [pltpu.SemaphoreType.DMA for _ in range(NBUF)],
        ],
        compiler_params=pltpu.CompilerParams(use_tc_tiling_on_sc=False,
                                             needs_layout_passes=False),
    )
    return fn(src, dst, *gs, bits, zg)


def _epilogue_body(a0, a1, a2, a3, a4, a5, a6, a7, q,
                   feat_ref, h0_ref, hg_ref, ipo, imod,
                   gw1, gb1, gw2, gb2, mw1, mb1, mw2, mb2,
                   ow1, ob1, ow2, ob2, out_ref):
    s64 = jnp.concatenate([a0[...], a1[...], a2[...], a3[...]], axis=1)
    t64 = jnp.concatenate([a4[...], a5[...], a6[...], a7[...]], axis=1)
    ng = t64 / (s64 + 1e-9)
    qq = q[...]
    sb = qq[:, 0:1]
    nb = qq[:, 1:2] / (sb + 1e-9)
    feat = feat_ref[...]

    xg = jnp.concatenate([ng, feat], axis=1)
    hgate = jnp.dot(_leaky(jnp.dot(xg, gw1[...],
                                   preferred_element_type=jnp.float32)
                           + gb1[...]), gw2[...],
                    preferred_element_type=jnp.float32) + gb2[...]
    xm = jnp.concatenate([ng, nb, feat], axis=1)
    hmod = jnp.dot(_leaky(jnp.dot(xm, mw1[...],
                                  preferred_element_type=jnp.float32)
                          + mb1[...]), mw2[...],
                   preferred_element_type=jnp.float32) + mb2[...]
    not_po = ipo[...] != 1
    hgate = jnp.where(not_po, jnp.maximum(hgate, 0.0), hgate)
    hmod = jnp.where(not_po, jnp.maximum(hmod, 0.0), hmod)
    h = jnp.where(imod[...] == 1, hmod, hgate)
    h = jnp.where(sb == 0, h0_ref[...], h)
    xo = jnp.concatenate([h, hg_ref[...]], axis=1)
    out_ref[...] = jnp.dot(_leaky(jnp.dot(xo, ow1[...],
                                          preferred_element_type=jnp.float32)
                                  + ob1[...]), ow2[...],
                           preferred_element_type=jnp.float32) + ob2[...]


def _epilogue(os_, q, feat, h0, hg, is_po, is_module,
              gate_w1, gate_b1, gate_w2, gate_b2,
              mod_w1, mod_b1, mod_w2, mod_b2,
              out_w1, out_b1, out_w2, out_b2):
    bg = pl.BlockSpec((BN, GW), lambda i: (i, 0))
    full2 = lambda a: pl.BlockSpec(a.shape, lambda i: (0, 0))
    ws = [gate_w1, gate_b1.reshape(1, -1), gate_w2, gate_b2.reshape(1, -1),
          mod_w1, mod_b1.reshape(1, -1), mod_w2, mod_b2.reshape(1, -1),
          out_w1, out_b1.reshape(1, -1), out_w2, out_b2.reshape(1, -1)]
    return pl.pallas_call(
        _epilogue_body,
        grid=(NB,),
        in_specs=[bg] * 8
        + [pl.BlockSpec((BN, GW), lambda i: (i, 0)),
                  pl.BlockSpec((BN, F), lambda i: (i, 0)),
                  pl.BlockSpec((BN, H), lambda i: (i, 0)),
                  pl.BlockSpec((BN, H), lambda i: (i, 0)),
                  pl.BlockSpec((BN, 1), lambda i: (i, 0)),
                  pl.BlockSpec((BN, 1), lambda i: (i, 0))]
        + [full2(a) for a in ws],
        out_specs=pl.BlockSpec((BN, 1), lambda i: (i, 0)),
        out_shape=jax.ShapeDtypeStruct((N, 1), jnp.float32),
    )(*os_, q, feat, h0, hg,
      is_po.reshape(N, 1), is_module.reshape(N, 1), *ws)


def kernel(feat, delay, bit_position, pi_w1, pi_b1, pi_w2, pi_b2,
           gate_w1, gate_b1, gate_w2, gate_b2,
           mod_w1, mod_b1, mod_w2, mod_b2,
           glob_w1, glob_b1, glob_w2, glob_b2,
           out_w1, out_b1, out_w2, out_b2,
           edge_index, is_po, is_module):
    src = edge_index[0]
    dst = edge_index[1]

    h0, hg, gmx = _node_embed(delay, pi_w1, pi_b1, pi_w2, pi_b2,
                              glob_w1, glob_b1, glob_w2, glob_b2)
    gs = _tables(h0, gmx)
    *os_, q = _edge_pass(src, dst, gs, bit_position)

    return _epilogue(os_, q, feat, h0, hg, is_po, is_module,
                     gate_w1, gate_b1, gate_w2, gate_b2,
                     mod_w1, mod_b1, mod_w2, mod_b2,
                     out_w1, out_b1, out_w2, out_b2)
